# Initial kernel scaffold; baseline (speedup 1.0000x reference)
import jax
import jax.numpy as jnp
from jax.experimental import pallas as pl


def kernel(inputs, table, W1, b1, W2, b2):
    raise NotImplementedError("write your pallas kernel here")


# speedup vs baseline: 6.6251x; 6.6251x over previous
import functools

import jax
import jax.numpy as jnp
from jax import lax
from jax.experimental import pallas as pl
from jax.experimental.pallas import tpu as pltpu
from jax.experimental.pallas import tpu_sc as plsc

VOCAB = 1000000
EMBED = 32
HIDDEN = 64
OUT = 2
BATCH = 16384
FIELDS = 26

ROW_BLK = 8192

NC = 2
NS = 16
NW = NC * NS
TOTAL = BATCH * FIELDS
PER_TILE = TOTAL // NW
CHUNK = 128
NCHUNK = PER_TILE // CHUNK
GROUP = 8


def _mlp_body(tbl_ref, w1_ref, b1_ref, w2_ref, b2_ref, out_ref):
    x = tbl_ref[...]
    h = jnp.dot(x, w1_ref[...], preferred_element_type=jnp.float32) + b1_ref[...]
    h = jnp.maximum(h, 0.0)
    logits = jnp.dot(h, w2_ref[...], preferred_element_type=jnp.float32) + b2_ref[...]
    m = jnp.max(logits, axis=-1, keepdims=True)
    e = jnp.exp(logits - m)
    out_ref[...] = e / jnp.sum(e, axis=-1, keepdims=True)


def _vocab_mlp(table, W1, b1, W2, b2):
    return pl.pallas_call(
        _mlp_body,
        grid=(pl.cdiv(VOCAB, ROW_BLK),),
        in_specs=[
            pl.BlockSpec((ROW_BLK, EMBED), lambda i: (i, 0)),
            pl.BlockSpec((EMBED, HIDDEN), lambda i: (0, 0)),
            pl.BlockSpec((1, HIDDEN), lambda i: (0, 0)),
            pl.BlockSpec((HIDDEN, OUT), lambda i: (0, 0)),
            pl.BlockSpec((1, OUT), lambda i: (0, 0)),
        ],
        out_specs=pl.BlockSpec((ROW_BLK, OUT), lambda i: (i, 0)),
        out_shape=jax.ShapeDtypeStruct((VOCAB, OUT), jnp.float32),
    )(table, W1, b1.reshape(1, HIDDEN), W2, b2.reshape(1, OUT))


@functools.partial(
    pl.kernel,
    out_type=jax.ShapeDtypeStruct((NW, NCHUNK, CHUNK, OUT), jnp.float32),
    mesh=plsc.VectorSubcoreMesh(core_axis_name="c", subcore_axis_name="s"),
    scratch_types=[
        pltpu.VMEM((NCHUNK, CHUNK), jnp.int32),
        pltpu.VMEM((NCHUNK, CHUNK, OUT), jnp.float32),
        pltpu.SemaphoreType.DMA,
    ],
    compiler_params=pltpu.CompilerParams(use_tc_tiling_on_sc=False),
)
def _sc_gather(tbl_hbm, idx_hbm, out_hbm, idx_v, rows_v, sem):
    wid = lax.axis_index("s") * NC + lax.axis_index("c")
    pltpu.sync_copy(idx_hbm.at[wid], idx_v)

    def body(g, carry):
        base = g * GROUP
        for b in range(GROUP):
            pltpu.async_copy(tbl_hbm.at[idx_v.at[base + b]], rows_v.at[base + b], sem)
        for b in range(GROUP):
            pltpu.make_async_copy(
                tbl_hbm.at[idx_v.at[base + b]], rows_v.at[base + b], sem
            ).wait()
        return carry

    lax.fori_loop(0, NCHUNK // GROUP, body, 0)
    pltpu.sync_copy(rows_v, out_hbm.at[wid])


def kernel(inputs, table, W1, b1, W2, b2):
    probs = _vocab_mlp(table, W1, b1, W2, b2)
    idx = inputs.astype(jnp.int32).reshape(NW, NCHUNK, CHUNK)
    out = _sc_gather(probs, idx)
    return out.reshape(BATCH, FIELDS, OUT)


# speedup vs baseline: 6.6285x; 1.0005x over previous
import functools

import jax
import jax.numpy as jnp
from jax import lax
from jax.experimental import pallas as pl
from jax.experimental.pallas import tpu as pltpu
from jax.experimental.pallas import tpu_sc as plsc

VOCAB = 1000000
EMBED = 32
HIDDEN = 64
OUT = 2
BATCH = 16384
FIELDS = 26

ROW_BLK = 8192

NC = 2
NS = 16
NW = NC * NS
TOTAL = BATCH * FIELDS
PER_TILE = TOTAL // NW
CHUNK = 128
NCHUNK = PER_TILE // CHUNK
GROUP = 8


def _mlp_body(tbl_ref, w1_ref, b1_ref, w2_ref, b2_ref, out_ref):
    x = tbl_ref[...]
    h = jnp.dot(x, w1_ref[...], preferred_element_type=jnp.float32) + b1_ref[...]
    h = jnp.maximum(h, 0.0)
    logits = jnp.dot(h, w2_ref[...], preferred_element_type=jnp.float32) + b2_ref[...]
    m = jnp.max(logits, axis=-1, keepdims=True)
    e = jnp.exp(logits - m)
    out_ref[...] = e / jnp.sum(e, axis=-1, keepdims=True)


def _vocab_mlp(table, W1, b1, W2, b2):
    return pl.pallas_call(
        _mlp_body,
        grid=(pl.cdiv(VOCAB, ROW_BLK),),
        in_specs=[
            pl.BlockSpec((ROW_BLK, EMBED), lambda i: (i, 0)),
            pl.BlockSpec((EMBED, HIDDEN), lambda i: (0, 0)),
            pl.BlockSpec((1, HIDDEN), lambda i: (0, 0)),
            pl.BlockSpec((HIDDEN, OUT), lambda i: (0, 0)),
            pl.BlockSpec((1, OUT), lambda i: (0, 0)),
        ],
        out_specs=pl.BlockSpec((ROW_BLK, OUT), lambda i: (i, 0)),
        out_shape=jax.ShapeDtypeStruct((VOCAB, OUT), jnp.float32),
    )(table, W1, b1.reshape(1, HIDDEN), W2, b2.reshape(1, OUT))


@functools.partial(
    pl.kernel,
    out_type=jax.ShapeDtypeStruct((NW, NCHUNK, CHUNK, OUT), jnp.float32),
    mesh=plsc.VectorSubcoreMesh(core_axis_name="c", subcore_axis_name="s"),
    scratch_types=[
        pltpu.VMEM((NCHUNK, CHUNK), jnp.int32),
        pltpu.VMEM((NCHUNK, CHUNK, OUT), jnp.float32),
        pltpu.SemaphoreType.DMA,
    ],
    compiler_params=pltpu.CompilerParams(use_tc_tiling_on_sc=False),
)
def _sc_gather(tbl_hbm, idx_hbm, out_hbm, idx_v, rows_v, sem):
    wid = lax.axis_index("s") * NC + lax.axis_index("c")
    pltpu.sync_copy(idx_hbm.at[wid], idx_v)

    def fire(g):
        base = g * GROUP
        for b in range(GROUP):
            pltpu.async_copy(tbl_hbm.at[idx_v.at[base + b]], rows_v.at[base + b], sem)

    def drain(g):
        base = g * GROUP
        for b in range(GROUP):
            pltpu.make_async_copy(
                tbl_hbm.at[idx_v.at[base + b]], rows_v.at[base + b], sem
            ).wait()

    n_groups = NCHUNK // GROUP
    fire(0)

    def body(g, carry):
        fire(g + 1)
        drain(g)
        return carry

    lax.fori_loop(0, n_groups - 1, body, 0)
    drain(n_groups - 1)
    pltpu.sync_copy(rows_v, out_hbm.at[wid])


def kernel(inputs, table, W1, b1, W2, b2):
    probs = _vocab_mlp(table, W1, b1, W2, b2)
    idx = inputs.astype(jnp.int32).reshape(NW, NCHUNK, CHUNK)
    out = _sc_gather(probs, idx)
    return out.reshape(BATCH, FIELDS, OUT)
